# baseline (speedup 1.0000x reference)
import functools

import jax
import jax.numpy as jnp
from jax import lax
from jax.experimental import pallas as pl
from jax.experimental.pallas import tpu as pltpu
from jax.experimental.pallas import tpu_sc as plsc


def _glyph_gather(ids8, table, B, L, S):
    info = plsc.get_sparse_core_info()
    NC, NS = info.num_cores, info.num_subcores
    NW = NC * NS
    assert B % NW == 0 and (B // NW) % 2 == 0
    bpw = B // NW
    NQ = 4
    H = S // NQ
    RPI = L * S * S // 128
    QR = H * L * S // 128

    mesh = plsc.VectorSubcoreMesh(core_axis_name="c", subcore_axis_name="s")

    @functools.partial(
        pl.kernel,
        mesh=mesh,
        out_type=jax.ShapeDtypeStruct((B * S * L * S // 128, 128),
                                      jnp.float32),
        compiler_params=pltpu.CompilerParams(use_tc_tiling_on_sc=True),
        scratch_types=[
            pltpu.VMEM((bpw * L * 8 // 128, 128), jnp.int32),
            pltpu.VMEM((2, RPI, 128), jnp.float32),
            pltpu.VMEM((NQ, QR, 128), jnp.float32),
            pltpu.SemaphoreType.DMA,
            pltpu.SemaphoreType.DMA,
            pltpu.SemaphoreType.DMA,
        ],
    )
    def k(ids_hbm, table_hbm, out_hbm, ids_v, t_v, u_v, gsem0, gsem1, wsem):
        wid = lax.axis_index("s") * NC + lax.axis_index("c")
        base = wid * bpw
        nrows = bpw * L * 8 // 128
        pltpu.sync_copy(ids_hbm.at[pl.ds(wid * nrows, nrows)], ids_v)

        def fire(i, buf, sem):
            for j in range(2):
                pltpu.async_copy(
                    table_hbm.at[ids_v.at[i * 2 + j]],
                    t_v.at[buf, pl.ds(j * 128, 128)], sem)

        def gather_drained(buf, sem):
            pltpu.make_async_copy(
                table_hbm.at[pl.ds(0, RPI)], t_v.at[buf], sem).wait()

        def writes_drained(i):
            @pl.when(i > 0)
            def _():
                for q in range(NQ):
                    pltpu.make_async_copy(
                        out_hbm.at[pl.ds(0, QR)], u_v.at[q], wsem).wait()

        def emit(i, buf):
            writes_drained(i)
            for q in range(NQ):
                def tbody(hr, _):
                    r = q * H + hr
                    for cc in range(S // 16):
                        e = r * S + cc * 16
                        for l in range(L):
                            o = hr * L * S + l * S + cc * 16
                            u_v[q, o // 128, pl.ds(o % 128, 16)] = t_v[
                                buf, l * 8 + e // 128, pl.ds(e % 128, 16)]
                    return 0
                lax.fori_loop(0, H, tbody, 0)
                pltpu.async_copy(
                    u_v.at[q],
                    out_hbm.at[pl.ds((base + i) * RPI + q * QR, QR)], wsem)

        def loop(ii, carry):
            i0 = ii * 2
            fire(i0 + 1, 1, gsem1)
            gather_drained(0, gsem0)
            emit(i0, 0)

            @pl.when(ii + 1 < bpw // 2)
            def _():
                fire(i0 + 2, 0, gsem0)
            gather_drained(1, gsem1)
            emit(i0 + 1, 1)
            return carry

        fire(0, 0, gsem0)
        lax.fori_loop(0, bpw // 2, loop, 0)
        for q in range(NQ):
            pltpu.make_async_copy(
                out_hbm.at[pl.ds(0, QR)], u_v.at[q], wsem).wait()

    return k(ids8, table)


def kernel(inputs, embeddings):
    B, L = inputs.shape
    V, S, S2, C = embeddings.shape
    idx = inputs.astype(jnp.int32)
    ids8 = (idx[:, :, None] * 8 +
            lax.broadcasted_iota(jnp.int32, (1, 1, 8), 2))
    ids8 = ids8.reshape(B * L * 8 // 128, 128)
    table = embeddings.reshape(V * 8, (S * S2) // 8)
    out = _glyph_gather(ids8, table, B, L, S)
    return out.reshape(B, S, L * S2, 1)


# speedup vs baseline: 1.8622x; 1.8622x over previous
import functools

import jax
import jax.numpy as jnp
from jax import lax
from jax.experimental import pallas as pl
from jax.experimental.pallas import tpu as pltpu
from jax.experimental.pallas import tpu_sc as plsc


def _glyph_gather(ids8, table, B, L, S):
    info = plsc.get_sparse_core_info()
    NC, NS = info.num_cores, info.num_subcores
    NW = NC * NS
    assert B % NW == 0 and (B // NW) % 2 == 0
    bpw = B // NW
    NQ = 4
    H = S // NQ
    RPI = L * S * S // 128
    QR = H * L * S // 128

    mesh = plsc.VectorSubcoreMesh(core_axis_name="c", subcore_axis_name="s")

    @functools.partial(
        pl.kernel,
        mesh=mesh,
        out_type=jax.ShapeDtypeStruct((B * S * L * S // 128, 128),
                                      jnp.float32),
        compiler_params=pltpu.CompilerParams(use_tc_tiling_on_sc=True),
        scratch_types=[
            pltpu.VMEM((bpw * L // 128, 128), jnp.int32),
            pltpu.VMEM((2, L, S * S), jnp.float32),
            pltpu.VMEM((NQ, QR, 128), jnp.float32),
            pltpu.SemaphoreType.DMA,
            pltpu.SemaphoreType.DMA,
            pltpu.SemaphoreType.DMA,
        ],
    )
    def k(ids_hbm, table_hbm, out_hbm, ids_v, t_v, u_v, gsem0, gsem1, wsem):
        wid = lax.axis_index("s") * NC + lax.axis_index("c")
        base = wid * bpw
        nrows = bpw * L // 128
        pltpu.sync_copy(ids_hbm.at[pl.ds(wid * nrows, nrows)], ids_v)

        def fire(i, buf, sem):
            p = i * L
            pltpu.async_copy(
                table_hbm.at[ids_v.at[p // 128, pl.ds(p % 128, L)]],
                t_v.at[buf], sem)

        def gather_drained(buf, sem):
            pltpu.make_async_copy(
                table_hbm.at[pl.ds(0, L)], t_v.at[buf], sem).wait()

        def writes_drained(i):
            @pl.when(i > 0)
            def _():
                for q in range(NQ):
                    pltpu.make_async_copy(
                        out_hbm.at[pl.ds(0, QR)], u_v.at[q], wsem).wait()

        def emit(i, buf):
            writes_drained(i)
            for q in range(NQ):
                def tbody(hr, _):
                    r = q * H + hr
                    for cc in range(S // 16):
                        e = r * S + cc * 16
                        for l in range(L):
                            o = hr * L * S + l * S + cc * 16
                            u_v[q, o // 128, pl.ds(o % 128, 16)] = t_v[
                                buf, l, pl.ds(e, 16)]
                    return 0
                lax.fori_loop(0, H, tbody, 0)
                pltpu.async_copy(
                    u_v.at[q],
                    out_hbm.at[pl.ds((base + i) * RPI + q * QR, QR)], wsem)

        def loop(ii, carry):
            i0 = ii * 2
            fire(i0 + 1, 1, gsem1)
            gather_drained(0, gsem0)
            emit(i0, 0)

            @pl.when(ii + 1 < bpw // 2)
            def _():
                fire(i0 + 2, 0, gsem0)
            gather_drained(1, gsem1)
            emit(i0 + 1, 1)
            return carry

        fire(0, 0, gsem0)
        lax.fori_loop(0, bpw // 2, loop, 0)
        for q in range(NQ):
            pltpu.make_async_copy(
                out_hbm.at[pl.ds(0, QR)], u_v.at[q], wsem).wait()

    return k(ids8, table)


def kernel(inputs, embeddings):
    B, L = inputs.shape
    V, S, S2, C = embeddings.shape
    idx = inputs.astype(jnp.int32)
    ids = idx.reshape(B * L // 128, 128)
    table = embeddings.reshape(V, S * S2)
    out = _glyph_gather(ids, table, B, L, S)
    return out.reshape(B, S, L * S2, 1)
